# Initial kernel scaffold; baseline (speedup 1.0000x reference)
import jax
import jax.numpy as jnp
from jax.experimental import pallas as pl


def kernel(sent, cenb_w, cemb_w, negwords):
    raise NotImplementedError("write your pallas kernel here")


# speedup vs baseline: 3.8975x; 3.8975x over previous
import functools
import math

import jax
import jax.numpy as jnp
from jax import lax
from jax.experimental import pallas as pl
from jax.experimental.pallas import tpu as pltpu
from jax.experimental.pallas import tpu_sc as plsc

VSIZE = 1000000
ESIZE = 16
SENTLEN = 50
RAD = 5
NSAMPL = 5
BATCH = 4096

NC, NS = 2, 16
NW = NC * NS
CHUNK = 128
WINDOW = 16

ROWS_W = BATCH * SENTLEN // NW
NCH = ROWS_W // CHUNK
NROWS_NEG_W = BATCH * NSAMPL // NW
NCH_NEG = NROWS_NEG_W // CHUNK


def _sc_gather_body(cenb_hbm, cemb_hbm, sent_hbm, negw_hbm,
                    cen_out, con_out, neg_out,
                    idx_v, nidx_v, rows_v, sem):
    wid = lax.axis_index("s") * NC + lax.axis_index("c")
    pltpu.sync_copy(sent_hbm.at[wid], idx_v)
    pltpu.sync_copy(negw_hbm.at[wid], nidx_v)

    def gather_to(table_hbm, out_hbm, idx_ref, nch):
        def body(j, carry):
            pltpu.async_copy(table_hbm.at[idx_ref.at[j]],
                             rows_v.at[pl.ds(j * CHUNK, CHUNK)], sem)

            @pl.when(j >= WINDOW)
            def _():
                pltpu.make_async_copy(table_hbm.at[pl.ds(0, CHUNK)],
                                      rows_v.at[pl.ds(0, CHUNK)], sem).wait()

            return carry

        lax.fori_loop(0, nch, body, 0)
        tail = min(nch, WINDOW) * CHUNK
        pltpu.make_async_copy(table_hbm.at[pl.ds(0, tail)],
                              rows_v.at[pl.ds(0, tail)], sem).wait()
        pltpu.sync_copy(rows_v.at[pl.ds(0, nch * CHUNK)], out_hbm.at[wid])

    gather_to(cenb_hbm, cen_out, idx_v, NCH)
    gather_to(cemb_hbm, con_out, idx_v, NCH)
    gather_to(cemb_hbm, neg_out, nidx_v, NCH_NEG)


@functools.cache
def _make_sc_gather():
    return pl.kernel(
        _sc_gather_body,
        out_type=[
            jax.ShapeDtypeStruct((NW, ROWS_W, ESIZE), jnp.float32),
            jax.ShapeDtypeStruct((NW, ROWS_W, ESIZE), jnp.float32),
            jax.ShapeDtypeStruct((NW, NROWS_NEG_W, ESIZE), jnp.float32),
        ],
        mesh=plsc.VectorSubcoreMesh(core_axis_name="c", subcore_axis_name="s",
                                    num_cores=NC, num_subcores=NS),
        scratch_types=[
            pltpu.VMEM((NCH, CHUNK), jnp.int32),
            pltpu.VMEM((NCH_NEG, CHUNK), jnp.int32),
            pltpu.VMEM((ROWS_W, ESIZE), jnp.float32),
            pltpu.SemaphoreType.DMA,
        ],
        compiler_params=pltpu.CompilerParams(use_tc_tiling_on_sc=False),
    )

BB = 256


def _softplus(x):
    return jnp.log1p(jnp.exp(-jnp.abs(x))) + jnp.maximum(x, 0.0)


def _tc_loss_body(cen_ref, con_ref, neg_ref, out_ref):
    i = pl.program_id(0)
    cen = cen_ref[...]
    con = con_ref[...]
    neg = neg_ref[...]
    pos = jnp.float32(0.0)
    for d in range(1, RAD + 1):
        a = jnp.sum(cen[:, :SENTLEN - d, :] * con[:, d:, :], axis=2)
        b = jnp.sum(cen[:, d:, :] * con[:, :SENTLEN - d, :], axis=2)
        pos += jnp.sum(_softplus(-a)) + jnp.sum(_softplus(-b))
    negsum = jnp.float32(0.0)
    for n in range(NSAMPL):
        s = jnp.sum(cen * neg[:, n:n + 1, :], axis=2)
        negsum += jnp.sum(_softplus(s))
    val = (pos / (BATCH * SENTLEN * SENTLEN)
           + negsum / (BATCH * SENTLEN * NSAMPL))

    @pl.when(i == 0)
    def _():
        out_ref[...] = jnp.zeros((1, 1), jnp.float32)

    out_ref[...] = out_ref[...] + val


_tc_loss = pl.pallas_call(
    _tc_loss_body,
    grid=(BATCH // BB,),
    in_specs=[
        pl.BlockSpec((BB, SENTLEN, ESIZE), lambda i: (i, 0, 0)),
        pl.BlockSpec((BB, SENTLEN, ESIZE), lambda i: (i, 0, 0)),
        pl.BlockSpec((BB, NSAMPL, ESIZE), lambda i: (i, 0, 0)),
    ],
    out_specs=pl.BlockSpec((1, 1), lambda i: (0, 0)),
    out_shape=jax.ShapeDtypeStruct((1, 1), jnp.float32),
)


def kernel(sent, cenb_w, cemb_w, negwords):
    sent_r = sent.astype(jnp.int32).reshape(NW, NCH, CHUNK)
    negw_r = negwords.astype(jnp.int32).reshape(NW, NCH_NEG, CHUNK)
    cen_g, con_g, neg_g = _make_sc_gather()(cenb_w, cemb_w, sent_r, negw_r)
    cen = cen_g.reshape(BATCH, SENTLEN, ESIZE)
    con = con_g.reshape(BATCH, SENTLEN, ESIZE)
    neg = neg_g.reshape(BATCH, NSAMPL, ESIZE)
    out = _tc_loss(cen, con, neg)
    return out[0, 0]


# speedup vs baseline: 9.7357x; 2.4980x over previous
import functools
import math

import jax
import jax.numpy as jnp
from jax import lax
from jax.experimental import pallas as pl
from jax.experimental.pallas import tpu as pltpu
from jax.experimental.pallas import tpu_sc as plsc

VSIZE = 1000000
ESIZE = 16
SENTLEN = 50
RAD = 5
NSAMPL = 5
BATCH = 4096

NC, NS = 2, 16
NW = NC * NS
CHUNK = 128
WINDOW = 16

ROWS_W = BATCH * SENTLEN // NW
NCH = ROWS_W // CHUNK
NROWS_NEG_W = BATCH * NSAMPL // NW
NCH_NEG = NROWS_NEG_W // CHUNK


def _sc_gather_body(cenb_hbm, cemb_hbm, sent_hbm, negw_hbm,
                    cen_out, con_out, neg_out,
                    idx_v, nidx_v, rows_v, sem):
    wid = lax.axis_index("s") * NC + lax.axis_index("c")
    pltpu.sync_copy(sent_hbm.at[wid], idx_v)
    pltpu.sync_copy(negw_hbm.at[wid], nidx_v)

    def gather_to(table_hbm, out_hbm, idx_ref, nch):
        def body(j, carry):
            pltpu.async_copy(table_hbm.at[idx_ref.at[j]],
                             rows_v.at[pl.ds(j * CHUNK, CHUNK)], sem)

            @pl.when(j >= WINDOW)
            def _():
                pltpu.make_async_copy(table_hbm.at[pl.ds(0, CHUNK)],
                                      rows_v.at[pl.ds(0, CHUNK)], sem).wait()

            return carry

        lax.fori_loop(0, nch, body, 0)
        tail = min(nch, WINDOW) * CHUNK
        pltpu.make_async_copy(table_hbm.at[pl.ds(0, tail)],
                              rows_v.at[pl.ds(0, tail)], sem).wait()
        pltpu.sync_copy(rows_v.at[pl.ds(0, nch * CHUNK)], out_hbm.at[wid])

    gather_to(cenb_hbm, cen_out, idx_v, NCH)
    gather_to(cemb_hbm, con_out, idx_v, NCH)
    gather_to(cemb_hbm, neg_out, nidx_v, NCH_NEG)


@functools.cache
def _make_sc_gather():
    return pl.kernel(
        _sc_gather_body,
        out_type=[
            jax.ShapeDtypeStruct((NW, ROWS_W, ESIZE), jnp.float32),
            jax.ShapeDtypeStruct((NW, ROWS_W, ESIZE), jnp.float32),
            jax.ShapeDtypeStruct((NW, NROWS_NEG_W, ESIZE), jnp.float32),
        ],
        mesh=plsc.VectorSubcoreMesh(core_axis_name="c", subcore_axis_name="s",
                                    num_cores=NC, num_subcores=NS),
        scratch_types=[
            pltpu.VMEM((NCH, CHUNK), jnp.int32),
            pltpu.VMEM((NCH_NEG, CHUNK), jnp.int32),
            pltpu.VMEM((ROWS_W, ESIZE), jnp.float32),
            pltpu.SemaphoreType.DMA,
        ],
        compiler_params=pltpu.CompilerParams(use_tc_tiling_on_sc=False),
    )

BB = 512
LE = SENTLEN * ESIZE
NE = NSAMPL * ESIZE


def _softplus(x):
    return jnp.log1p(jnp.exp(-jnp.abs(x))) + jnp.maximum(x, 0.0)


def _tc_loss_body(cen_ref, con_ref, neg_ref, out_ref):
    i = pl.program_id(0)
    cen = cen_ref[...]
    con = con_ref[...]
    neg = neg_ref[...]
    S = (lax.broadcasted_iota(jnp.int32, (LE, SENTLEN), 0) // ESIZE
         == lax.broadcasted_iota(jnp.int32, (LE, SENTLEN), 1)
         ).astype(jnp.float32)
    Trep = (lax.broadcasted_iota(jnp.int32, (ESIZE, LE), 0)
            == lax.broadcasted_iota(jnp.int32, (ESIZE, LE), 1) % ESIZE
            ).astype(jnp.float32)
    pos = jnp.zeros((), jnp.float32)
    for d in range(1, RAD + 1):
        w = LE - ESIZE * d
        p1 = cen[:, :w] * con[:, ESIZE * d:]
        p2 = cen[:, ESIZE * d:] * con[:, :w]
        s1 = jnp.dot(p1, S[:w, :SENTLEN - d],
                     preferred_element_type=jnp.float32)
        s2 = jnp.dot(p2, S[:w, :SENTLEN - d],
                     preferred_element_type=jnp.float32)
        pos += jnp.sum(_softplus(-s1)) + jnp.sum(_softplus(-s2))
    negsum = jnp.zeros((), jnp.float32)
    for n in range(NSAMPL):
        nb = jnp.dot(neg[:, ESIZE * n:ESIZE * (n + 1)], Trep,
                     preferred_element_type=jnp.float32)
        s = jnp.dot(cen * nb, S, preferred_element_type=jnp.float32)
        negsum += jnp.sum(_softplus(s))
    val = (pos / (BATCH * SENTLEN * SENTLEN)
           + negsum / (BATCH * SENTLEN * NSAMPL))

    @pl.when(i == 0)
    def _():
        out_ref[...] = jnp.zeros((1, 1), jnp.float32)

    out_ref[...] = out_ref[...] + val


_tc_loss = pl.pallas_call(
    _tc_loss_body,
    grid=(BATCH // BB,),
    in_specs=[
        pl.BlockSpec((BB, LE), lambda i: (i, 0)),
        pl.BlockSpec((BB, LE), lambda i: (i, 0)),
        pl.BlockSpec((BB, NE), lambda i: (i, 0)),
    ],
    out_specs=pl.BlockSpec((1, 1), lambda i: (0, 0)),
    out_shape=jax.ShapeDtypeStruct((1, 1), jnp.float32),
)


def kernel(sent, cenb_w, cemb_w, negwords):
    sent_r = sent.astype(jnp.int32).reshape(NW, NCH, CHUNK)
    negw_r = negwords.astype(jnp.int32).reshape(NW, NCH_NEG, CHUNK)
    cen_g, con_g, neg_g = _make_sc_gather()(cenb_w, cemb_w, sent_r, negw_r)
    cen = cen_g.reshape(BATCH, LE)
    con = con_g.reshape(BATCH, LE)
    neg = neg_g.reshape(BATCH, NE)
    out = _tc_loss(cen, con, neg)
    return out[0, 0]
